# baseline (speedup 1.0000x reference)
import functools

import jax
import jax.numpy as jnp
from jax import lax
from jax.experimental import pallas as pl
from jax.experimental.pallas import tpu as pltpu
from jax.experimental.pallas import tpu_sc as plsc

N = 10000
C = 256
T = 4
E = 160000
CW = 128
NCH = (T * C) // CW
EB = 125
NTILES = 16
NCORES = 2
NPAD = 10240
ROWS_PER_TILE = NPAD // NTILES
NB = 1000
EPS = 1e-12

_sc_mesh = functools.partial(
    plsc.VectorSubcoreMesh, core_axis_name="c", subcore_axis_name="s")


def _deg_body(dst_hbm, ones_hbm, zeros_hbm, out_hbm, ones_v, zeros_v, idx_v, acc_sh, sem):
    cidx = lax.axis_index("c")
    sidx = lax.axis_index("s")
    pltpu.sync_copy(ones_hbm, ones_v)
    pltpu.sync_copy(zeros_hbm, zeros_v)
    pltpu.sync_copy(dst_hbm.at[cidx, sidx], idx_v)
    for k in range(ROWS_PER_TILE // 32):
        pltpu.sync_copy(zeros_v, acc_sh.at[pl.ds(sidx * ROWS_PER_TILE + k * 32, 32)])
    plsc.subcore_barrier()

    def body(j, carry):
        pltpu.sync_copy(ones_v, acc_sh.at[idx_v.at[j]], add=True)
        return carry

    lax.fori_loop(0, E // (NCORES * NTILES * EB), body, 0)
    plsc.subcore_barrier()
    pltpu.sync_copy(acc_sh.at[pl.ds(sidx * ROWS_PER_TILE, ROWS_PER_TILE)],
                    out_hbm.at[cidx, pl.ds(sidx * ROWS_PER_TILE, ROWS_PER_TILE)])


def _make_deg_kernel():
    return pl.kernel(
        _deg_body,
        mesh=_sc_mesh(),
        out_type=jax.ShapeDtypeStruct((NCORES, NPAD, CW), jnp.float32),
        scratch_types=[
            pltpu.VMEM((EB, CW), jnp.float32),
            pltpu.VMEM((32, CW), jnp.float32),
            pltpu.VMEM((E // (NCORES * NTILES * EB), EB), jnp.int32),
            pltpu.VMEM_SHARED((NPAD, CW), jnp.float32),
            pltpu.SemaphoreType.DMA,
        ],
    )


def _mm_scale_body(s_ref, part_ref, w_ref, q8_ref, deg_ref):
    deg = part_ref[0, :, 0:1] + part_ref[1, :, 0:1] + 1.0
    deg_ref[...] = deg
    dinv = lax.rsqrt(jnp.maximum(deg, EPS))
    w = w_ref[...]
    for t in range(T):
        xwt = lax.dot_general(s_ref[t], w, (((1,), (0,)), ((), ())),
                              preferred_element_type=jnp.float32)
        qt = xwt * dinv
        q8_ref[2 * t] = qt[:, :CW]
        q8_ref[2 * t + 1] = qt[:, CW:]


def _mm_scale_call(s_seq, part, W):
    return pl.pallas_call(
        _mm_scale_body,
        grid=(N // NB,),
        in_specs=[
            pl.BlockSpec((T, NB, C), lambda nb: (0, nb, 0)),
            pl.BlockSpec((NCORES, NB, CW), lambda nb: (0, nb, 0)),
            pl.BlockSpec((C, C), lambda nb: (0, 0)),
        ],
        out_specs=[
            pl.BlockSpec((NCH, NB, CW), lambda nb: (0, nb, 0)),
            pl.BlockSpec((NB, 1), lambda nb: (nb, 0)),
        ],
        out_shape=[
            jax.ShapeDtypeStruct((NCH, N, CW), jnp.float32),
            jax.ShapeDtypeStruct((N, 1), jnp.float32),
        ],
    )(s_seq, part, W)


EBP = 128
NBATCH = 82


def _agg_body(q_hbm, srcoff_hbm, dst_hbm, zeros_hbm, out_hbm,
              src20, src21, dst20, dst21, rows0, rows1, zeros_v, acc_sh,
              isrc0, isrc1, idst0, idst1, gsem0, gsem1):
    cidx = lax.axis_index("c")
    sidx = lax.axis_index("s")
    src2 = [src20, src21]
    dst2 = [dst20, dst21]
    rows = [rows0, rows1]
    isrc = [isrc0, isrc1]
    idst = [idst0, idst1]
    gsem = [gsem0, gsem1]
    pltpu.sync_copy(zeros_hbm, zeros_v)
    for cc in range(NCH // NCORES):
        chunk = cidx * (NCH // NCORES) + cc
        for k in range(ROWS_PER_TILE // 32):
            pltpu.sync_copy(
                zeros_v, acc_sh.at[pl.ds(sidx * ROWS_PER_TILE + k * 32, 32)])
        plsc.subcore_barrier()

        pltpu.async_copy(srcoff_hbm.at[chunk, sidx, 0], src20, isrc0).wait()
        pltpu.async_copy(dst_hbm.at[sidx, 0], dst20, idst0)
        pltpu.async_copy(q_hbm.at[src20], rows0, gsem0)
        pltpu.async_copy(srcoff_hbm.at[chunk, sidx, 1], src21, isrc1)
        pltpu.async_copy(dst_hbm.at[sidx, 1], dst21, idst1)

        def body(jj, carry):
            for b in (0, 1):
                nb2 = 1 - b
                j = 2 * jj + b
                pltpu.make_async_copy(
                    srcoff_hbm.at[chunk, sidx, j + 1], src2[nb2], isrc[nb2]).wait()
                pltpu.async_copy(q_hbm.at[src2[nb2]], rows[nb2], gsem[nb2])
                pltpu.make_async_copy(q_hbm.at[src2[b]], rows[b], gsem[b]).wait()
                pltpu.make_async_copy(dst_hbm.at[sidx, j], dst2[b], idst[b]).wait()
                pltpu.sync_copy(rows[b], acc_sh.at[dst2[b]], add=True)
                pltpu.async_copy(srcoff_hbm.at[chunk, sidx, j + 2], src2[b], isrc[b])
                pltpu.async_copy(dst_hbm.at[sidx, j + 2], dst2[b], idst[b])
            return carry

        lax.fori_loop(0, (NBATCH - 2) // 2, body, 0)
        pltpu.make_async_copy(q_hbm.at[src20], rows0, gsem0).wait()
        pltpu.make_async_copy(dst_hbm.at[sidx, 0], dst20, idst0).wait()
        pltpu.make_async_copy(srcoff_hbm.at[chunk, sidx, 0], src21, isrc1).wait()
        pltpu.make_async_copy(dst_hbm.at[sidx, 0], dst21, idst1).wait()
        plsc.subcore_barrier()
        pltpu.sync_copy(
            acc_sh.at[pl.ds(sidx * ROWS_PER_TILE, ROWS_PER_TILE)],
            out_hbm.at[chunk, pl.ds(sidx * ROWS_PER_TILE, ROWS_PER_TILE)])


def _make_agg_kernel():
    return pl.kernel(
        _agg_body,
        mesh=_sc_mesh(),
        out_type=jax.ShapeDtypeStruct((NCH, NPAD, CW), jnp.float32),
        scratch_types=[
            pltpu.VMEM((EBP,), jnp.int32),
            pltpu.VMEM((EBP,), jnp.int32),
            pltpu.VMEM((EBP,), jnp.int32),
            pltpu.VMEM((EBP,), jnp.int32),
            pltpu.VMEM((EBP, CW), jnp.float32),
            pltpu.VMEM((EBP, CW), jnp.float32),
            pltpu.VMEM((32, CW), jnp.float32),
            pltpu.VMEM_SHARED((NPAD, CW), jnp.float32),
            pltpu.SemaphoreType.DMA,
            pltpu.SemaphoreType.DMA,
            pltpu.SemaphoreType.DMA,
            pltpu.SemaphoreType.DMA,
            pltpu.SemaphoreType.DMA,
            pltpu.SemaphoreType.DMA,
        ],
    )


def _final_body(agg_ref, q_ref, deg_ref, z_ref, o_ref, znew_ref):
    dinv = lax.rsqrt(jnp.maximum(deg_ref[...], EPS))
    xs = []
    for t in range(T):
        aggt = jnp.concatenate([agg_ref[2 * t], agg_ref[2 * t + 1]], axis=1)
        qt = jnp.concatenate([q_ref[2 * t], q_ref[2 * t + 1]], axis=1)
        xs.append((aggt + qt) * dinv)
    y = (xs[0] + xs[1] + xs[2] + xs[3]) * (0.1 / T)
    z = z_ref[...]
    for t in range(T):
        u = z + (xs[t] + y - z) * 0.5
        o = jnp.where(u > 1.0, 1.0, 0.0)
        z = u - o
        o_ref[t] = o
    znew_ref[...] = z


def _final_call(agg, q8, deg, z_seq):
    return pl.pallas_call(
        _final_body,
        grid=(N // NB,),
        in_specs=[
            pl.BlockSpec((NCH, NB, CW), lambda nb: (0, nb, 0)),
            pl.BlockSpec((NCH, NB, CW), lambda nb: (0, nb, 0)),
            pl.BlockSpec((NB, 1), lambda nb: (nb, 0)),
            pl.BlockSpec((NB, C), lambda nb: (nb, 0)),
        ],
        out_specs=[
            pl.BlockSpec((T, NB, C), lambda nb: (0, nb, 0)),
            pl.BlockSpec((NB, C), lambda nb: (nb, 0)),
        ],
        out_shape=[
            jax.ShapeDtypeStruct((T, N, C), jnp.float32),
            jax.ShapeDtypeStruct((N, C), jnp.float32),
        ],
    )(agg, q8, deg, z_seq)


def kernel(s_seq, z_seq, edge_index, W):
    ei = edge_index.astype(jnp.int32)
    src, dst = ei[0], ei[1]
    tiles_deg = NCORES * NTILES
    dst_deg = dst.reshape(NCORES, NTILES, E // (tiles_deg * EB), EB)

    pad_len = NTILES * (NBATCH - 2) * EBP - E
    srcp = jnp.concatenate([src, jnp.zeros((pad_len,), jnp.int32)])
    dstp = jnp.concatenate([dst, jnp.full((pad_len,), NPAD - 1, jnp.int32)])
    srcp = srcp.reshape(NTILES, NBATCH - 2, EBP)
    dstp = dstp.reshape(NTILES, NBATCH - 2, EBP)
    srcp = jnp.pad(srcp, ((0, 0), (0, 2), (0, 0)))
    dstp = jnp.pad(dstp, ((0, 0), (0, 2), (0, 0)), constant_values=NPAD - 1)
    srcoff = srcp[None] + (jnp.arange(NCH, dtype=jnp.int32) * N)[:, None, None, None]

    ones_c = jnp.ones((EB, CW), jnp.float32)
    zeros_w = jnp.zeros((32, CW), jnp.float32)

    part = _make_deg_kernel()(dst_deg, ones_c, zeros_w)
    q8, deg = _mm_scale_call(s_seq, part, W)
    q_flat = q8.reshape(NCH * N, CW)
    agg = _make_agg_kernel()(q_flat, srcoff, dstp, zeros_w)
    o_seq, z_new = _final_call(agg, q8, deg, z_seq)
    return (o_seq, z_new)


# speedup vs baseline: 1.2763x; 1.2763x over previous
import functools

import jax
import jax.numpy as jnp
from jax import lax
from jax.experimental import pallas as pl
from jax.experimental.pallas import tpu as pltpu
from jax.experimental.pallas import tpu_sc as plsc

N = 10000
C = 256
T = 4
E = 160000
CW = 128
NCH = (T * C) // CW
EB = 125
NTILES = 16
NCORES = 2
NPAD = 10112
ROWS_PER_TILE = NPAD // NTILES
NB = 1000
EPS = 1e-12

_sc_mesh = functools.partial(
    plsc.VectorSubcoreMesh, core_axis_name="c", subcore_axis_name="s")


def _deg_body(dst_hbm, ones_hbm, zeros_hbm, out_hbm, ones_v, idx_v, acc_sh, sem):
    cidx = lax.axis_index("c")
    sidx = lax.axis_index("s")
    pltpu.sync_copy(ones_hbm, ones_v)
    pltpu.sync_copy(dst_hbm.at[cidx, sidx], idx_v)
    pltpu.sync_copy(zeros_hbm, acc_sh.at[pl.ds(sidx * ROWS_PER_TILE, ROWS_PER_TILE)])
    plsc.subcore_barrier()

    def body(j, carry):
        pltpu.sync_copy(ones_v, acc_sh.at[idx_v.at[j]], add=True)
        return carry

    lax.fori_loop(0, E // (NCORES * NTILES * EB), body, 0)
    plsc.subcore_barrier()
    pltpu.sync_copy(acc_sh.at[pl.ds(sidx * ROWS_PER_TILE, ROWS_PER_TILE)],
                    out_hbm.at[cidx, pl.ds(sidx * ROWS_PER_TILE, ROWS_PER_TILE)])


def _make_deg_kernel():
    return pl.kernel(
        _deg_body,
        mesh=_sc_mesh(),
        out_type=jax.ShapeDtypeStruct((NCORES, NPAD, CW), jnp.float32),
        scratch_types=[
            pltpu.VMEM((EB, CW), jnp.float32),
            pltpu.VMEM((E // (NCORES * NTILES * EB), EB), jnp.int32),
            pltpu.VMEM_SHARED((NPAD, CW), jnp.float32),
            pltpu.SemaphoreType.DMA,
        ],
    )


def _mm_scale_body(s_ref, part_ref, w_ref, q8_ref, deg_ref):
    deg = part_ref[0, :, 0:1] + part_ref[1, :, 0:1] + 1.0
    deg_ref[...] = deg
    dinv = lax.rsqrt(jnp.maximum(deg, EPS))
    w = w_ref[...]
    for t in range(T):
        xwt = lax.dot_general(s_ref[t], w, (((1,), (0,)), ((), ())),
                              preferred_element_type=jnp.float32)
        qt = xwt * dinv
        q8_ref[2 * t] = qt[:, :CW]
        q8_ref[2 * t + 1] = qt[:, CW:]


def _mm_scale_call(s_seq, part, W):
    return pl.pallas_call(
        _mm_scale_body,
        grid=(N // NB,),
        in_specs=[
            pl.BlockSpec((T, NB, C), lambda nb: (0, nb, 0)),
            pl.BlockSpec((NCORES, NB, CW), lambda nb: (0, nb, 0)),
            pl.BlockSpec((C, C), lambda nb: (0, 0)),
        ],
        out_specs=[
            pl.BlockSpec((NCH, NB, CW), lambda nb: (0, nb, 0)),
            pl.BlockSpec((NB, 1), lambda nb: (nb, 0)),
        ],
        out_shape=[
            jax.ShapeDtypeStruct((NCH, N, CW), jnp.float32),
            jax.ShapeDtypeStruct((N, 1), jnp.float32),
        ],
    )(s_seq, part, W)


EBP = 104
NRB = 97
NBT = NRB + 1


def _agg_body(q_hbm, srcoff_hbm, dst_hbm, zeros_hbm, out_hbm,
              sidx_v, dst_v, rows0, rows1, acc_sh,
              gsem0, gsem1, ssem0, ssem1):
    cidx = lax.axis_index("c")
    sidx = lax.axis_index("s")
    rows = [rows0, rows1]
    gsem = [gsem0, gsem1]
    ssem = [ssem0, ssem1]
    pltpu.sync_copy(dst_hbm.at[sidx], dst_v)
    for cc in range(NCH // NCORES):
        chunk = cidx * (NCH // NCORES) + cc
        pltpu.sync_copy(srcoff_hbm.at[chunk, sidx], sidx_v)
        pltpu.sync_copy(
            zeros_hbm, acc_sh.at[pl.ds(sidx * ROWS_PER_TILE, ROWS_PER_TILE)])
        plsc.subcore_barrier()

        pltpu.async_copy(q_hbm.at[sidx_v.at[pl.ds(0, EBP)]], rows0, gsem0).wait()
        pltpu.async_copy(rows0, acc_sh.at[dst_v.at[0]], ssem0, add=True)
        pltpu.async_copy(q_hbm.at[sidx_v.at[pl.ds(EBP, EBP)]], rows1, gsem1)

        def body(jj, carry):
            for b in (1, 0):
                nb2 = 1 - b
                j = 2 * jj + 1 + (1 - b)
                pltpu.make_async_copy(
                    q_hbm.at[sidx_v.at[pl.ds(j * EBP, EBP)]], rows[b], gsem[b]).wait()
                pltpu.async_copy(rows[b], acc_sh.at[dst_v.at[j]], ssem[b], add=True)
                pltpu.make_async_copy(rows[nb2], acc_sh.at[dst_v.at[j]], ssem[nb2]).wait()
                pltpu.async_copy(
                    q_hbm.at[sidx_v.at[pl.ds((j + 1) * EBP, EBP)]], rows[nb2], gsem[nb2])
            return carry

        lax.fori_loop(0, (NRB - 1) // 2, body, 0)
        pltpu.make_async_copy(rows0, acc_sh.at[dst_v.at[0]], ssem0).wait()
        pltpu.make_async_copy(q_hbm.at[sidx_v.at[pl.ds(0, EBP)]], rows1, gsem1).wait()
        plsc.subcore_barrier()
        pltpu.sync_copy(
            acc_sh.at[pl.ds(sidx * ROWS_PER_TILE, ROWS_PER_TILE)],
            out_hbm.at[chunk, pl.ds(sidx * ROWS_PER_TILE, ROWS_PER_TILE)])


def _make_agg_kernel():
    return pl.kernel(
        _agg_body,
        mesh=_sc_mesh(),
        out_type=jax.ShapeDtypeStruct((NCH, NPAD, CW), jnp.float32),
        scratch_types=[
            pltpu.VMEM((NBT * EBP,), jnp.int32),
            pltpu.VMEM((NRB, EBP), jnp.int32),
            pltpu.VMEM((EBP, CW), jnp.float32),
            pltpu.VMEM((EBP, CW), jnp.float32),
            pltpu.VMEM_SHARED((NPAD, CW), jnp.float32),
            pltpu.SemaphoreType.DMA,
            pltpu.SemaphoreType.DMA,
            pltpu.SemaphoreType.DMA,
            pltpu.SemaphoreType.DMA,
        ],
    )


def _final_body(agg_ref, q_ref, deg_ref, z_ref, o_ref, znew_ref):
    dinv = lax.rsqrt(jnp.maximum(deg_ref[...], EPS))
    xs = []
    for t in range(T):
        aggt = jnp.concatenate([agg_ref[2 * t], agg_ref[2 * t + 1]], axis=1)
        qt = jnp.concatenate([q_ref[2 * t], q_ref[2 * t + 1]], axis=1)
        xs.append((aggt + qt) * dinv)
    y = (xs[0] + xs[1] + xs[2] + xs[3]) * (0.1 / T)
    z = z_ref[...]
    for t in range(T):
        u = z + (xs[t] + y - z) * 0.5
        o = jnp.where(u > 1.0, 1.0, 0.0)
        z = u - o
        o_ref[t] = o
    znew_ref[...] = z


def _final_call(agg, q8, deg, z_seq):
    return pl.pallas_call(
        _final_body,
        grid=(N // NB,),
        in_specs=[
            pl.BlockSpec((NCH, NB, CW), lambda nb: (0, nb, 0)),
            pl.BlockSpec((NCH, NB, CW), lambda nb: (0, nb, 0)),
            pl.BlockSpec((NB, 1), lambda nb: (nb, 0)),
            pl.BlockSpec((NB, C), lambda nb: (nb, 0)),
        ],
        out_specs=[
            pl.BlockSpec((T, NB, C), lambda nb: (0, nb, 0)),
            pl.BlockSpec((NB, C), lambda nb: (nb, 0)),
        ],
        out_shape=[
            jax.ShapeDtypeStruct((T, N, C), jnp.float32),
            jax.ShapeDtypeStruct((N, C), jnp.float32),
        ],
    )(agg, q8, deg, z_seq)


def kernel(s_seq, z_seq, edge_index, W):
    ei = edge_index.astype(jnp.int32)
    src, dst = ei[0], ei[1]
    tiles_deg = NCORES * NTILES
    dst_deg = dst.reshape(NCORES, NTILES, E // (tiles_deg * EB), EB)

    pad_len = NTILES * NRB * EBP - E
    srcp = jnp.concatenate([src, jnp.zeros((pad_len,), jnp.int32)])
    dstp = jnp.concatenate([dst, jnp.full((pad_len,), NPAD - 1, jnp.int32)])
    srcp = srcp.reshape(NTILES, NRB, EBP)
    dstp = dstp.reshape(NTILES, NRB, EBP)
    srcp = jnp.pad(srcp, ((0, 0), (0, 1), (0, 0))).reshape(NTILES, NBT * EBP)
    srcoff = srcp[None] + (jnp.arange(NCH, dtype=jnp.int32) * N)[:, None, None]

    ones_c = jnp.ones((EB, CW), jnp.float32)
    zeros_row = jnp.zeros((ROWS_PER_TILE, CW), jnp.float32)

    part = _make_deg_kernel()(dst_deg, ones_c, zeros_row)
    q8, deg = _mm_scale_call(s_seq, part, W)
    q_flat = q8.reshape(NCH * N, CW)
    agg = _make_agg_kernel()(q_flat, srcoff, dstp, zeros_row)
    o_seq, z_new = _final_call(agg, q8, deg, z_seq)
    return (o_seq, z_new)
